# Initial kernel scaffold; baseline (speedup 1.0000x reference)
import functools
import math

import jax
import jax.numpy as jnp
from jax import lax
from jax.experimental import pallas as pl
from jax.experimental.pallas import tpu as pltpu
from jax.experimental.pallas import tpu_sc as plsc

_N = 4096
_V = 32000
_PAD = 0
_SMOOTH = 0.1
_EPS = _SMOOTH / (_V - 2)
_CONF = 1.0 - _SMOOTH
_C = _CONF * math.log(_CONF) + _SMOOTH * math.log(_EPS)

_SC_CORES = 2
_SC_SUBCORES = 16
_SC_LANES = 16
_NW = _SC_CORES * _SC_SUBCORES
_BPW = _N // _NW

_BC = 1280


@functools.partial(
    pl.kernel,
    mesh=plsc.VectorSubcoreMesh(core_axis_name="c", subcore_axis_name="s"),
    out_type=jax.ShapeDtypeStruct((_N,), jnp.float32),
    scratch_types=[
        pltpu.VMEM((_BPW,), jnp.int32),
        pltpu.VMEM((_BPW,), jnp.int32),
        pltpu.VMEM((_BPW,), jnp.float32),
        pltpu.SemaphoreType.DMA,
    ],
)
def _sc_gather(x_hbm, tgt_hbm, g_hbm, tgt_v, idx_v, g_v, sem):
    wid = lax.axis_index("s") * _SC_CORES + lax.axis_index("c")
    base = wid * _BPW
    pltpu.sync_copy(tgt_hbm.at[pl.ds(base, _BPW)], tgt_v)

    @pl.loop(0, _BPW, step=_SC_LANES)
    def _(k):
        rows = (base + k) + lax.iota(jnp.int32, _SC_LANES)
        idx_v[pl.ds(k, _SC_LANES)] = rows * _V + tgt_v[pl.ds(k, _SC_LANES)]

    pltpu.async_copy(x_hbm.at[idx_v], g_v, sem).wait()
    pltpu.sync_copy(g_v, g_hbm.at[pl.ds(base, _BPW)])


def _tc_body(tgt_ref, g_ref, x_ref, out_ref):
    j = pl.program_id(0)
    nonpad = (tgt_ref[...] != _PAD).astype(jnp.float32)
    rowsums = jnp.sum(x_ref[...], axis=1, keepdims=True)

    @pl.when(j == 0)
    def _():
        x0 = x_ref[:, 0:1]
        head = nonpad * (_C + _EPS * x0 - (_CONF - _EPS) * g_ref[...])
        out_ref[0, 0] = jnp.sum(head)

    out_ref[0, 0] += -_EPS * jnp.sum(nonpad * rowsums)


def kernel(x, tgt):
    tgt = tgt.astype(jnp.int32)
    g = _sc_gather(x.reshape(-1), tgt)
    total = pl.pallas_call(
        _tc_body,
        grid=(_V // _BC,),
        in_specs=[
            pl.BlockSpec((_N, 1), lambda j: (0, 0)),
            pl.BlockSpec((_N, 1), lambda j: (0, 0)),
            pl.BlockSpec((_N, _BC), lambda j: (0, j)),
        ],
        out_specs=pl.BlockSpec((1, 1), lambda j: (0, 0)),
        out_shape=jax.ShapeDtypeStruct((1, 1), jnp.float32),
    )(tgt.reshape(_N, 1), g.reshape(_N, 1), x)
    return total[0, 0]


# speedup vs baseline: 2.5253x; 2.5253x over previous
import functools
import math

import jax
import jax.numpy as jnp
from jax import lax
from jax.experimental import pallas as pl
from jax.experimental.pallas import tpu as pltpu
from jax.experimental.pallas import tpu_sc as plsc

_N = 4096
_V = 32000
_PAD = 0
_SMOOTH = 0.1
_EPS = _SMOOTH / (_V - 2)
_CONF = 1.0 - _SMOOTH
_C = _CONF * math.log(_CONF) + _SMOOTH * math.log(_EPS)

_SC_CORES = 2
_SC_SUBCORES = 16
_SC_LANES = 16
_NW = _SC_CORES * _SC_SUBCORES
_BPW = _N // _NW

_BC = 1280


def _sc_gather_body(x_hbm, tgt_hbm, g_hbm, tgt_v, idx_v, g_v, sem):
    wid = lax.axis_index("s") * _SC_CORES + lax.axis_index("c")
    base = wid * _BPW
    pltpu.sync_copy(tgt_hbm.at[pl.ds(base, _BPW)], tgt_v)

    @pl.loop(0, _BPW, step=_SC_LANES)
    def _(k):
        rows = (base + k) + lax.iota(jnp.int32, _SC_LANES)
        idx_v[pl.ds(k, _SC_LANES)] = rows * _V + tgt_v[pl.ds(k, _SC_LANES)]

    pltpu.async_copy(x_hbm.at[idx_v], g_v, sem).wait()
    pltpu.sync_copy(g_v, g_hbm.at[pl.ds(base, _BPW)])


def _tc_body(tgt_ref, g_ref, x_ref, out_ref):
    j = pl.program_id(0)
    nonpad = (tgt_ref[...] != _PAD).astype(jnp.float32)
    rowsums = jnp.sum(x_ref[...], axis=1, keepdims=True)

    @pl.when(j == 0)
    def _():
        x0 = x_ref[:, 0:1]
        head = nonpad * (_C + _EPS * x0 - (_CONF - _EPS) * g_ref[...])
        out_ref[...] = jnp.sum(head).reshape(1, 1)

    out_ref[...] += (-_EPS * jnp.sum(nonpad * rowsums)).reshape(1, 1)


def _make_sc_gather():
    return pl.kernel(
        _sc_gather_body,
        mesh=plsc.VectorSubcoreMesh(
            core_axis_name="c", subcore_axis_name="s",
            num_cores=_SC_CORES, num_subcores=_SC_SUBCORES,
        ),
        out_type=jax.ShapeDtypeStruct((_N,), jnp.float32),
        scratch_types=[
            pltpu.VMEM((_BPW,), jnp.int32),
            pltpu.VMEM((_BPW,), jnp.int32),
            pltpu.VMEM((_BPW,), jnp.float32),
            pltpu.SemaphoreType.DMA,
        ],
    )


def kernel(x, tgt):
    tgt = tgt.astype(jnp.int32)
    g = _make_sc_gather()(x.reshape(-1), tgt)
    total = pl.pallas_call(
        _tc_body,
        grid=(_V // _BC,),
        in_specs=[
            pl.BlockSpec((_N, 1), lambda j: (0, 0)),
            pl.BlockSpec((_N, 1), lambda j: (0, 0)),
            pl.BlockSpec((_N, _BC), lambda j: (0, j)),
        ],
        out_specs=pl.BlockSpec((1, 1), lambda j: (0, 0)),
        out_shape=jax.ShapeDtypeStruct((1, 1), jnp.float32),
    )(tgt.reshape(_N, 1), g.reshape(_N, 1), x)
    return total[0, 0]


# speedup vs baseline: 2.5385x; 1.0052x over previous
import functools
import math

import jax
import jax.numpy as jnp
from jax import lax
from jax.experimental import pallas as pl
from jax.experimental.pallas import tpu as pltpu
from jax.experimental.pallas import tpu_sc as plsc

_N = 4096
_V = 32000
_PAD = 0
_SMOOTH = 0.1
_EPS = _SMOOTH / (_V - 2)
_CONF = 1.0 - _SMOOTH
_C = _CONF * math.log(_CONF) + _SMOOTH * math.log(_EPS)

_SC_CORES = 2
_SC_SUBCORES = 16
_SC_LANES = 16
_NW = _SC_CORES * _SC_SUBCORES
_BPW = _N // _NW

_BR = 128


def _sc_gather_body(x_hbm, tgt_hbm, g_hbm, tgt_v, idx_v, g_v, sem):
    wid = lax.axis_index("s") * _SC_CORES + lax.axis_index("c")
    base = wid * _BPW
    pltpu.sync_copy(tgt_hbm.at[pl.ds(base, _BPW)], tgt_v)

    @pl.loop(0, _BPW, step=_SC_LANES)
    def _(k):
        rows = (base + k) + lax.iota(jnp.int32, _SC_LANES)
        idx_v[pl.ds(k, _SC_LANES)] = rows * _V + tgt_v[pl.ds(k, _SC_LANES)]

    pltpu.async_copy(x_hbm.at[idx_v], g_v, sem).wait()
    pltpu.sync_copy(g_v, g_hbm.at[pl.ds(base, _BPW)])


def _tc_body(tgt_ref, g_ref, x_ref, out_ref):
    j = pl.program_id(0)

    @pl.when(j == 0)
    def _():
        out_ref[...] = jnp.zeros((1, 1), jnp.float32)

    nonpad = (tgt_ref[...] != _PAD).astype(jnp.float32)
    rowsums = jnp.sum(x_ref[...], axis=1, keepdims=True)
    x0 = x_ref[:, 0:1]
    per_row = _C + _EPS * x0 - (_CONF - _EPS) * g_ref[...] - _EPS * rowsums
    out_ref[...] += jnp.sum(nonpad * per_row).reshape(1, 1)


def _make_sc_gather():
    return pl.kernel(
        _sc_gather_body,
        mesh=plsc.VectorSubcoreMesh(
            core_axis_name="c", subcore_axis_name="s",
            num_cores=_SC_CORES, num_subcores=_SC_SUBCORES,
        ),
        out_type=jax.ShapeDtypeStruct((_N,), jnp.float32),
        scratch_types=[
            pltpu.VMEM((_BPW,), jnp.int32),
            pltpu.VMEM((_BPW,), jnp.int32),
            pltpu.VMEM((_BPW,), jnp.float32),
            pltpu.SemaphoreType.DMA,
        ],
    )


def kernel(x, tgt):
    tgt = tgt.astype(jnp.int32)
    g = _make_sc_gather()(x.reshape(-1), tgt)
    total = pl.pallas_call(
        _tc_body,
        grid=(_N // _BR,),
        in_specs=[
            pl.BlockSpec((_BR, 1), lambda j: (j, 0)),
            pl.BlockSpec((_BR, 1), lambda j: (j, 0)),
            pl.BlockSpec((_BR, _V), lambda j: (j, 0)),
        ],
        out_specs=pl.BlockSpec((1, 1), lambda j: (0, 0)),
        out_shape=jax.ShapeDtypeStruct((1, 1), jnp.float32),
    )(tgt.reshape(_N, 1), g.reshape(_N, 1), x)
    return total[0, 0]


# speedup vs baseline: 2.5405x; 1.0008x over previous
import functools
import math

import jax
import jax.numpy as jnp
from jax import lax
from jax.experimental import pallas as pl
from jax.experimental.pallas import tpu as pltpu
from jax.experimental.pallas import tpu_sc as plsc

_N = 4096
_V = 32000
_PAD = 0
_SMOOTH = 0.1
_EPS = _SMOOTH / (_V - 2)
_CONF = 1.0 - _SMOOTH
_C = _CONF * math.log(_CONF) + _SMOOTH * math.log(_EPS)

_SC_CORES = 2
_SC_SUBCORES = 16
_SC_LANES = 16
_NW = _SC_CORES * _SC_SUBCORES
_BPW = _N // _NW

_BR = 32
_K = 4


def _sc_gather_body(x_hbm, tgt_hbm, g_hbm, tgt_v, idx_v, g_v, sem):
    wid = lax.axis_index("s") * _SC_CORES + lax.axis_index("c")
    base = wid * _BPW
    pltpu.sync_copy(tgt_hbm.at[pl.ds(base, _BPW)], tgt_v)

    @pl.loop(0, _BPW, step=_SC_LANES)
    def _(k):
        rows = (base + k) + lax.iota(jnp.int32, _SC_LANES)
        idx_v[pl.ds(k, _SC_LANES)] = rows * _V + tgt_v[pl.ds(k, _SC_LANES)]

    pltpu.async_copy(x_hbm.at[idx_v], g_v, sem).wait()
    pltpu.sync_copy(g_v, g_hbm.at[pl.ds(base, _BPW)])


def _tc_body(tgt_ref, g_ref, *refs):
    *x_refs, out_ref = refs
    j = pl.program_id(0)

    @pl.when(j == 0)
    def _():
        out_ref[...] = jnp.zeros((1, 1), jnp.float32)

    nonpad = (tgt_ref[...] != _PAD).astype(jnp.float32)
    acc = jnp.zeros((1, 1), jnp.float32)
    for k, x_ref in enumerate(x_refs):
        sl = slice(k * _BR, (k + 1) * _BR)
        rowsums = jnp.sum(x_ref[...], axis=1, keepdims=True)
        x0 = x_ref[:, 0:1]
        per_row = (_C + _EPS * x0 - (_CONF - _EPS) * g_ref[sl, :]
                   - _EPS * rowsums)
        acc += jnp.sum(nonpad[sl, :] * per_row).reshape(1, 1)
    out_ref[...] += acc


def _make_sc_gather():
    return pl.kernel(
        _sc_gather_body,
        mesh=plsc.VectorSubcoreMesh(
            core_axis_name="c", subcore_axis_name="s",
            num_cores=_SC_CORES, num_subcores=_SC_SUBCORES,
        ),
        out_type=jax.ShapeDtypeStruct((_N,), jnp.float32),
        scratch_types=[
            pltpu.VMEM((_BPW,), jnp.int32),
            pltpu.VMEM((_BPW,), jnp.int32),
            pltpu.VMEM((_BPW,), jnp.float32),
            pltpu.SemaphoreType.DMA,
        ],
    )


def kernel(x, tgt):
    tgt = tgt.astype(jnp.int32)
    g = _make_sc_gather()(x.reshape(-1), tgt)
    x_specs = [
        pl.BlockSpec((_BR, _V), functools.partial(lambda k, j: (j * _K + k, 0), k))
        for k in range(_K)
    ]
    total = pl.pallas_call(
        _tc_body,
        grid=(_N // (_BR * _K),),
        in_specs=[
            pl.BlockSpec((_K * _BR, 1), lambda j: (j, 0)),
            pl.BlockSpec((_K * _BR, 1), lambda j: (j, 0)),
            *x_specs,
        ],
        out_specs=pl.BlockSpec((1, 1), lambda j: (0, 0)),
        out_shape=jax.ShapeDtypeStruct((1, 1), jnp.float32),
    )(tgt.reshape(_N, 1), g.reshape(_N, 1), *([x] * _K))
    return total[0, 0]
